# Initial kernel scaffold; baseline (speedup 1.0000x reference)
import jax
import jax.numpy as jnp
from jax.experimental import pallas as pl


def kernel(x, edge_index, W1, b1, W2, b2, Wc, bc, codebook, Wf, bf, Wd1, bd1, Wd2, bd2):
    raise NotImplementedError("write your pallas kernel here")


# speedup vs baseline: 1.3744x; 1.3744x over previous
import functools

import jax
import jax.numpy as jnp
from jax.experimental import pallas as pl
from jax.experimental.pallas import tpu as pltpu

N = 10000
E = 160000
IN = 256
H = 512
K = 8192
CD = 256
CC = 0.25

MB = 400
KB = 512
NM = N // MB
NK = K // KB


def _vq_body(z_ref, cb_ref, out_ref, dmin_ref, amin_ref, cc_ref):
    k = pl.program_id(1)
    z = z_ref[...]
    cb = cb_ref[...]

    @pl.when(pl.program_id(0) == 0)
    def _():
        cc_ref[k, :] = jnp.sum(cb * cb, axis=1)

    zz = jnp.sum(z * z, axis=1, keepdims=True)
    prod = jax.lax.dot_general(z, cb, (((1,), (1,)), ((), ())),
                               preferred_element_type=jnp.float32)
    d = zz - 2.0 * prod + cc_ref[k, :][None, :]
    loc_min = jnp.min(d, axis=1)
    loc_arg = jnp.argmin(d, axis=1).astype(jnp.int32) + k * KB

    @pl.when(k == 0)
    def _():
        dmin_ref[...] = loc_min
        amin_ref[...] = loc_arg

    @pl.when(k > 0)
    def _():
        upd = loc_min < dmin_ref[...]
        amin_ref[...] = jnp.where(upd, loc_arg, amin_ref[...])
        dmin_ref[...] = jnp.minimum(loc_min, dmin_ref[...])

    @pl.when(k == NK - 1)
    def _():
        out_ref[0, 0, :] = amin_ref[...]


def _vq_argmin(z, codebook):
    out = pl.pallas_call(
        _vq_body,
        grid=(NM, NK),
        in_specs=[
            pl.BlockSpec((MB, CD), lambda m, k: (m, 0)),
            pl.BlockSpec((KB, CD), lambda m, k: (k, 0)),
        ],
        out_specs=pl.BlockSpec((1, 1, MB), lambda m, k: (m, 0, 0)),
        out_shape=jax.ShapeDtypeStruct((NM, 1, MB), jnp.int32),
        scratch_shapes=[
            pltpu.VMEM((MB,), jnp.float32),
            pltpu.VMEM((MB,), jnp.int32),
            pltpu.VMEM((NK, KB), jnp.float32),
        ],
    )(z, codebook)
    return out.reshape(N)


def _gcn_conv(x, si, di, dinv, W, b):
    h = x @ W
    norm = (dinv[si] * dinv[di])[:, None]
    msg = h[si] * norm
    out = jnp.zeros((N, W.shape[1]), dtype=jnp.float32).at[di].add(msg)
    out = out + h * (dinv * dinv)[:, None]
    return out + b


def kernel(x, edge_index, W1, b1, W2, b2, Wc, bc, codebook, Wf, bf, Wd1, bd1, Wd2, bd2):
    src = edge_index[0]
    dst = edge_index[1]
    deg = jnp.zeros((N,), dtype=jnp.float32).at[dst].add(1.0) + 1.0
    dinv = jax.lax.rsqrt(deg)

    z = jax.nn.relu(_gcn_conv(x, src, dst, dinv, W1, b1))
    z = jax.nn.relu(_gcn_conv(z, src, dst, dinv, W2, b2))
    z = z @ Wc + bc

    indices = _vq_argmin(z, codebook)
    z_q = codebook[indices]
    loss = (1.0 + CC) * jnp.mean((z_q - z) ** 2)

    h = z_q @ Wf + bf
    h = jax.nn.relu(_gcn_conv(h, src, dst, dinv, Wd1, bd1))
    x_recon = _gcn_conv(h, src, dst, dinv, Wd2, bd2)
    return (x_recon, loss, indices)


# speedup vs baseline: 3.1010x; 2.2563x over previous
import functools

import jax
import jax.numpy as jnp
from jax import lax
from jax.experimental import pallas as pl
from jax.experimental.pallas import tpu as pltpu
from jax.experimental.pallas import tpu_sc as plsc

N = 10000
E = 160000
IN = 256
H = 512
K = 8192
CD = 256
CC = 0.25

NC = 2
NS = 16
NP = 10240
NT = NP // NS

ER = E // 128
ERT = 80
ERLAST = ER - (NS - 1) * ERT

DRW = 40
DRLAST = ER - 31 * DRW


@functools.cache
def _mesh():
    return plsc.VectorSubcoreMesh(core_axis_name="c", subcore_axis_name="s")


@functools.cache
def _deg_sc_kernel():
    return functools.partial(
        pl.kernel, mesh=_mesh(),
        out_type=jax.ShapeDtypeStruct((NC * NP, 16), jnp.float32),
        scratch_types=[
            pltpu.VMEM((DRW, 1, 128), jnp.int32),
            pltpu.VMEM((128, 16), jnp.float32),
            pltpu.VMEM((NT, 16), jnp.float32),
            pltpu.VMEM_SHARED((NP, 16), jnp.float32),
        ],
    )(_deg_sc_body)


def _deg_sc_body(di_hbm, out_hbm, didx_v, ones_v, stage_v, hist_sh):
    c = lax.axis_index("c")
    s = lax.axis_index("s")
    w = c * NS + s

    def fill_ones(i, _):
        ones_v[i, :] = jnp.ones((16,), jnp.float32)
        return 0
    lax.fori_loop(0, 128, fill_ones, 0)

    def fill_zero(i, _):
        stage_v[i, :] = jnp.zeros((16,), jnp.float32)
        return 0
    lax.fori_loop(0, NT, fill_zero, 0)

    pltpu.sync_copy(stage_v, hist_sh.at[pl.ds(s * NT, NT)])
    plsc.subcore_barrier()

    @pl.when(w < 31)
    def _():
        pltpu.sync_copy(di_hbm.at[pl.ds(w * DRW, DRW)], didx_v)

    @pl.when(w == 31)
    def _():
        pltpu.sync_copy(di_hbm.at[pl.ds(31 * DRW, DRLAST)],
                        didx_v.at[pl.ds(0, DRLAST)])

    nr = jnp.where(w < 31, DRW, DRLAST)

    def body(j, _):
        pltpu.sync_copy(ones_v, hist_sh.at[didx_v.at[j, 0]], add=True)
        return 0
    lax.fori_loop(0, nr, body, 0)

    plsc.subcore_barrier()
    pltpu.sync_copy(hist_sh.at[pl.ds(s * NT, NT)], stage_v)
    pltpu.sync_copy(stage_v, out_hbm.at[pl.ds(c * NP + s * NT, NT)])


NH = 5120
NPH = NH + 8
DUMP = NH
NTH = NH // NS
EL = ERT * 8


@functools.cache
def _conv_sc_kernel():
    return functools.partial(
        pl.kernel, mesh=_mesh(),
        out_type=[jax.ShapeDtypeStruct((NP, 128), jnp.float32) for _ in range(4)],
        scratch_types=[
            pltpu.VMEM((ERT, 1, 128), jnp.int32),
            pltpu.VMEM((ERT, 1, 128), jnp.int32),
            pltpu.VMEM((ERT, 1, 128), jnp.int32),
            pltpu.VMEM((128, 128), jnp.float32),
            pltpu.VMEM((128, 128), jnp.float32),
            pltpu.VMEM_SHARED((NPH, 128), jnp.float32),
            pltpu.SemaphoreType.DMA,
        ],
    )(_conv_sc_body)


def _conv_sc_body(hp0, hp1, hp2, hp3, si3d, di3d, o0, o1, o2, o3,
                  sidx_v, didx_v, dmap_v, rows_v, zero_v, acc_sh, sem):
    hp = (hp0, hp1, hp2, hp3)
    outs = (o0, o1, o2, o3)

    c = lax.axis_index("c")
    s = lax.axis_index("s")

    def fz(i, _):
        for jj in range(8):
            zero_v[i, pl.ds(jj * 16, 16)] = jnp.zeros((16,), jnp.float32)
        return 0
    lax.fori_loop(0, 128, fz, 0)

    @pl.when(s < NS - 1)
    def _():
        pltpu.sync_copy(si3d.at[pl.ds(s * ERT, ERT)], sidx_v)
        pltpu.sync_copy(di3d.at[pl.ds(s * ERT, ERT)], didx_v)

    @pl.when(s == NS - 1)
    def _():
        pltpu.sync_copy(si3d.at[pl.ds((NS - 1) * ERT, ERLAST)],
                        sidx_v.at[pl.ds(0, ERLAST)])
        pltpu.sync_copy(di3d.at[pl.ds((NS - 1) * ERT, ERLAST)],
                        didx_v.at[pl.ds(0, ERLAST)])

    nr = jnp.where(s < NS - 1, ERT, ERLAST)

    def map_half(h):
        def mbody(i, _):
            row = i // 8
            lane = (i % 8) * 16
            v = didx_v[row, 0, pl.ds(lane, 16)]
            t = v - (h * NH)
            ok = (t >= 0) & (t < NH)
            dmap_v[row, 0, pl.ds(lane, 16)] = jnp.where(ok, t, DUMP)
            return 0
        lax.fori_loop(0, EL, mbody, 0)

    def run_pass(hp_g, out_g, h):
        for k in range(3):
            rcnt = min(128, NTH - k * 128)
            pltpu.sync_copy(zero_v.at[pl.ds(0, rcnt)],
                            acc_sh.at[pl.ds(s * NTH + k * 128, rcnt)])
        plsc.subcore_barrier()

        def body(j, _):
            pltpu.async_copy(hp_g.at[sidx_v.at[j, 0]], rows_v, sem).wait()
            pltpu.sync_copy(rows_v, acc_sh.at[dmap_v.at[j, 0]], add=True)
            return 0
        lax.fori_loop(0, nr, body, 0)

        plsc.subcore_barrier()
        for k in range(3):
            rcnt = min(128, NTH - k * 128)
            pltpu.sync_copy(acc_sh.at[pl.ds(s * NTH + k * 128, rcnt)],
                            rows_v.at[pl.ds(0, rcnt)])
            pltpu.sync_copy(rows_v.at[pl.ds(0, rcnt)],
                            out_g.at[pl.ds(h * NH + s * NTH + k * 128, rcnt)])
        plsc.subcore_barrier()

    for h in range(2):
        map_half(h)
        for p in range(2):
            @pl.when(c == 0)
            def _(p=p, h=h):
                run_pass(hp[p], outs[p], h)

            @pl.when(c == 1)
            def _(p=p, h=h):
                run_pass(hp[2 + p], outs[2 + p], h)


NPAD = 12288
ZR = NPAD // 128
ZRW = ZR // (NC * NS)
ZB = ZRW * 128


@functools.cache
def _zq_sc_kernel():
    return functools.partial(
        pl.kernel, mesh=_mesh(),
        out_type=jax.ShapeDtypeStruct((NPAD, CD), jnp.float32),
        scratch_types=[
            pltpu.VMEM((ZRW, 1, 128), jnp.int32),
            pltpu.VMEM((ZB, CD), jnp.float32),
            pltpu.SemaphoreType.DMA,
        ],
    )(_zq_sc_body)


def _zq_sc_body(cb_hbm, idx_hbm, out_hbm, idx_v, rows_v, sem):
    c = lax.axis_index("c")
    s = lax.axis_index("s")
    w = c * NS + s
    pltpu.sync_copy(idx_hbm.at[pl.ds(w * ZRW, ZRW)], idx_v)
    for j in range(ZRW):
        pltpu.async_copy(cb_hbm.at[idx_v.at[j, 0]],
                         rows_v.at[pl.ds(j * 128, 128)], sem).wait()
    pltpu.sync_copy(rows_v, out_hbm.at[pl.ds(w * ZB, ZB)])


MB = 400
KB = 512
NM = N // MB
NK = K // KB


def _vq_body(z_ref, cb_ref, out_ref, dmin_ref, amin_ref, cc_ref):
    k = pl.program_id(1)
    z = z_ref[...]
    cb = cb_ref[...]

    @pl.when(pl.program_id(0) == 0)
    def _():
        cc_ref[k, :] = jnp.sum(cb * cb, axis=1)

    zz = jnp.sum(z * z, axis=1, keepdims=True)
    prod = lax.dot_general(z, cb, (((1,), (1,)), ((), ())),
                           preferred_element_type=jnp.float32)
    d = zz - 2.0 * prod + cc_ref[k, :][None, :]
    loc_min = jnp.min(d, axis=1)
    loc_arg = jnp.argmin(d, axis=1).astype(jnp.int32) + k * KB

    @pl.when(k == 0)
    def _():
        dmin_ref[...] = loc_min
        amin_ref[...] = loc_arg

    @pl.when(k > 0)
    def _():
        upd = loc_min < dmin_ref[...]
        amin_ref[...] = jnp.where(upd, loc_arg, amin_ref[...])
        dmin_ref[...] = jnp.minimum(loc_min, dmin_ref[...])

    @pl.when(k == NK - 1)
    def _():
        out_ref[0, 0, :] = amin_ref[...]


def _vq_argmin(z, codebook):
    out = pl.pallas_call(
        _vq_body,
        grid=(NM, NK),
        in_specs=[
            pl.BlockSpec((MB, CD), lambda m, k: (m, 0)),
            pl.BlockSpec((KB, CD), lambda m, k: (k, 0)),
        ],
        out_specs=pl.BlockSpec((1, 1, MB), lambda m, k: (m, 0, 0)),
        out_shape=jax.ShapeDtypeStruct((NM, 1, MB), jnp.int32),
        scratch_shapes=[
            pltpu.VMEM((MB,), jnp.float32),
            pltpu.VMEM((MB,), jnp.int32),
            pltpu.VMEM((NK, KB), jnp.float32),
        ],
    )(z, codebook)
    return out.reshape(N)


MBLK = 1000
GRID = N // MBLK


def _full(shape):
    return pl.BlockSpec(shape, lambda i: tuple(0 for _ in shape))


def _rows(width):
    return pl.BlockSpec((MBLK, width), lambda i: (i, 0))


def _gout(width=128):
    return pl.BlockSpec((MBLK, width), lambda i: (i, 0))


def _gcn_combine(acc_refs, hp_refs, dinv, b):
    a = jnp.concatenate(
        [acc_refs[g][...] + hp_refs[g][...] for g in range(len(acc_refs))], axis=1)
    return jnp.maximum(a * dinv + b, 0.0)


def _enc1_body(x_ref, dinv_ref, w_ref, o0, o1, o2, o3):
    y = jnp.dot(x_ref[...], w_ref[...], preferred_element_type=jnp.float32)
    y = y * dinv_ref[...]
    for g, o in enumerate((o0, o1, o2, o3)):
        o[...] = y[:, g * 128:(g + 1) * 128]


def _enc1(x, dinv, W1):
    return pl.pallas_call(
        _enc1_body,
        grid=(GRID,),
        in_specs=[_rows(IN), _rows(1), _full((IN, H))],
        out_specs=[_gout() for _ in range(4)],
        out_shape=[jax.ShapeDtypeStruct((N, 128), jnp.float32) for _ in range(4)],
    )(x, dinv, W1)


def _mid_body(a0, a1, a2, a3, h0, h1, h2, h3, dinv_ref, b_ref, w_ref,
              o0, o1, o2, o3):
    z = _gcn_combine((a0, a1, a2, a3), (h0, h1, h2, h3), dinv_ref[...], b_ref[...])
    y = jnp.dot(z, w_ref[...], preferred_element_type=jnp.float32)
    y = y * dinv_ref[...]
    for g, o in enumerate((o0, o1, o2, o3)):
        o[...] = y[:, g * 128:(g + 1) * 128]


def _mid(accs, hps, dinv, b, W):
    return pl.pallas_call(
        _mid_body,
        grid=(GRID,),
        in_specs=[_gout() for _ in range(8)] + [_rows(1), _full((1, H)), _full((H, H))],
        out_specs=[_gout() for _ in range(4)],
        out_shape=[jax.ShapeDtypeStruct((N, 128), jnp.float32) for _ in range(4)],
    )(*accs, *hps, dinv, b, W)


def _enc3_body(a0, a1, a2, a3, h0, h1, h2, h3, dinv_ref, b_ref, wc_ref, bc_ref, o):
    z2 = _gcn_combine((a0, a1, a2, a3), (h0, h1, h2, h3), dinv_ref[...], b_ref[...])
    o[...] = jnp.dot(z2, wc_ref[...], preferred_element_type=jnp.float32) + bc_ref[...]


def _enc3(accs, hps, dinv, b, Wc, bc):
    return pl.pallas_call(
        _enc3_body,
        grid=(GRID,),
        in_specs=[_gout() for _ in range(8)]
        + [_rows(1), _full((1, H)), _full((H, CD)), _full((1, CD))],
        out_specs=_rows(CD),
        out_shape=jax.ShapeDtypeStruct((N, CD), jnp.float32),
    )(*accs, *hps, dinv, b, Wc, bc)


def _dec1_body(zq_ref, z_ref, wf_ref, bf_ref, wd1_ref, dinv_ref,
               o0, o1, o2, o3, ls_ref):
    zq = zq_ref[...]
    h = jnp.dot(zq, wf_ref[...], preferred_element_type=jnp.float32) + bf_ref[...]
    y = jnp.dot(h, wd1_ref[...], preferred_element_type=jnp.float32)
    y = y * dinv_ref[...]
    for g, o in enumerate((o0, o1, o2, o3)):
        o[...] = y[:, g * 128:(g + 1) * 128]
    diff = zq - z_ref[...]
    part = jnp.sum(diff * diff)

    @pl.when(pl.program_id(0) == 0)
    def _():
        ls_ref[...] = part.reshape(1, 1)

    @pl.when(pl.program_id(0) > 0)
    def _():
        ls_ref[...] = ls_ref[...] + part.reshape(1, 1)


def _dec1(zq, z, Wf, bf, Wd1, dinv):
    return pl.pallas_call(
        _dec1_body,
        grid=(GRID,),
        in_specs=[_rows(CD), _rows(CD), _full((CD, H)), _full((1, H)),
                  _full((H, H)), _rows(1)],
        out_specs=[_gout() for _ in range(4)]
        + [pl.BlockSpec((1, 1), lambda i: (0, 0))],
        out_shape=[jax.ShapeDtypeStruct((N, 128), jnp.float32) for _ in range(4)]
        + [jax.ShapeDtypeStruct((1, 1), jnp.float32)],
    )(zq, z, Wf, bf, Wd1, dinv)


def _dec2_body(a0, a1, a2, a3, h0, h1, h2, h3, dinv_ref, b_ref, w_ref, o0, o1):
    h3v = _gcn_combine((a0, a1, a2, a3), (h0, h1, h2, h3), dinv_ref[...], b_ref[...])
    y = jnp.dot(h3v, w_ref[...], preferred_element_type=jnp.float32)
    y = y * dinv_ref[...]
    for g, o in enumerate((o0, o1)):
        o[...] = y[:, g * 128:(g + 1) * 128]


def _dec2(accs, hps, dinv, b, Wd2):
    return pl.pallas_call(
        _dec2_body,
        grid=(GRID,),
        in_specs=[_gout() for _ in range(8)] + [_rows(1), _full((1, H)), _full((H, IN))],
        out_specs=[_gout() for _ in range(2)],
        out_shape=[jax.ShapeDtypeStruct((N, 128), jnp.float32) for _ in range(2)],
    )(*accs, *hps, dinv, b, Wd2)


def _dec3_body(a0, a1, h0, h1, dinv_ref, b_ref, o):
    a = jnp.concatenate([a0[...] + h0[...], a1[...] + h1[...]], axis=1)
    o[...] = a * dinv_ref[...] + b_ref[...]


def _dec3(accs, hps, dinv, b):
    return pl.pallas_call(
        _dec3_body,
        grid=(GRID,),
        in_specs=[_gout() for _ in range(4)] + [_rows(1), _full((1, IN))],
        out_specs=_rows(IN),
        out_shape=jax.ShapeDtypeStruct((N, IN), jnp.float32),
    )(*accs, *hps, dinv, b)


def kernel(x, edge_index, W1, b1, W2, b2, Wc, bc, codebook, Wf, bf, Wd1, bd1, Wd2, bd2):
    si3d = edge_index[0].reshape(ER, 1, 128)
    di3d = edge_index[1].reshape(ER, 1, 128)

    conv = _conv_sc_kernel()

    ones_tbl = jnp.ones((N, 128), jnp.float32)
    deg_out = conv(ones_tbl, ones_tbl, ones_tbl, ones_tbl, si3d, di3d)
    deg = deg_out[0][:N, 0] + 1.0
    dinv = lax.rsqrt(deg).reshape(N, 1)

    hp1 = _enc1(x, dinv, W1)
    acc1 = conv(*hp1, si3d, di3d)
    hp2 = _mid(acc1, hp1, dinv, b1.reshape(1, H), W2)
    acc2 = conv(*hp2, si3d, di3d)
    z = _enc3(acc2, hp2, dinv, b2.reshape(1, H), Wc, bc.reshape(1, CD))

    indices = _vq_argmin(z, codebook)

    idxp = jnp.pad(indices, (0, NPAD - N)).reshape(ZR, 1, 128)
    z_q = _zq_sc_kernel()(codebook, idxp)[:N]

    *hp3, loss_sum = _dec1(z_q, z, Wf, bf.reshape(1, H), Wd1, dinv)
    loss = loss_sum[0, 0] * ((1.0 + CC) / (N * CD))

    acc3 = conv(*hp3, si3d, di3d)
    hp4 = _dec2(acc3, hp3, dinv, bd1.reshape(1, H), Wd2)
    acc4 = conv(hp4[0], hp4[1], hp4[0], hp4[1], si3d, di3d)[:2]
    x_recon = _dec3(acc4, hp4, dinv, bd2.reshape(1, IN))

    return (x_recon, loss, indices)


# speedup vs baseline: 4.2778x; 1.3795x over previous
import functools

import jax
import jax.numpy as jnp
from jax import lax
from jax.experimental import pallas as pl
from jax.experimental.pallas import tpu as pltpu
from jax.experimental.pallas import tpu_sc as plsc

N = 10000
E = 160000
IN = 256
H = 512
K = 8192
CD = 256
CC = 0.25

NC = 2
NS = 16
NP = 10240
NT = NP // NS

ER = E // 128
ERT = 80
ERLAST = ER - (NS - 1) * ERT

DRW = 40
DRLAST = ER - 31 * DRW


@functools.cache
def _mesh():
    return plsc.VectorSubcoreMesh(core_axis_name="c", subcore_axis_name="s")


@functools.cache
def _deg_sc_kernel():
    return functools.partial(
        pl.kernel, mesh=_mesh(),
        out_type=jax.ShapeDtypeStruct((NC * NP, 16), jnp.float32),
        scratch_types=[
            pltpu.VMEM((DRW, 1, 128), jnp.int32),
            pltpu.VMEM((128, 16), jnp.float32),
            pltpu.VMEM((NT, 16), jnp.float32),
            pltpu.VMEM_SHARED((NP, 16), jnp.float32),
        ],
    )(_deg_sc_body)


def _deg_sc_body(di_hbm, out_hbm, didx_v, ones_v, stage_v, hist_sh):
    c = lax.axis_index("c")
    s = lax.axis_index("s")
    w = c * NS + s

    def fill_ones(i, _):
        ones_v[i, :] = jnp.ones((16,), jnp.float32)
        return 0
    lax.fori_loop(0, 128, fill_ones, 0)

    def fill_zero(i, _):
        stage_v[i, :] = jnp.zeros((16,), jnp.float32)
        return 0
    lax.fori_loop(0, NT, fill_zero, 0)

    pltpu.sync_copy(stage_v, hist_sh.at[pl.ds(s * NT, NT)])
    plsc.subcore_barrier()

    @pl.when(w < 31)
    def _():
        pltpu.sync_copy(di_hbm.at[pl.ds(w * DRW, DRW)], didx_v)

    @pl.when(w == 31)
    def _():
        pltpu.sync_copy(di_hbm.at[pl.ds(31 * DRW, DRLAST)],
                        didx_v.at[pl.ds(0, DRLAST)])

    nr = jnp.where(w < 31, DRW, DRLAST)

    def body(j, _):
        pltpu.sync_copy(ones_v, hist_sh.at[didx_v.at[j, 0]], add=True)
        return 0
    lax.fori_loop(0, nr, body, 0)

    plsc.subcore_barrier()
    pltpu.sync_copy(hist_sh.at[pl.ds(s * NT, NT)], stage_v)
    pltpu.sync_copy(stage_v, out_hbm.at[pl.ds(c * NP + s * NT, NT)])


NH = 5120
NPH = NH + 8
DUMP = NH
NTH = NH // NS
EL = ERT * 8


@functools.cache
def _conv_sc_kernel():
    return functools.partial(
        pl.kernel, mesh=_mesh(),
        out_type=[jax.ShapeDtypeStruct((NP, 128), jnp.float32) for _ in range(4)],
        scratch_types=[
            pltpu.VMEM((ERT, 1, 128), jnp.int32),
            pltpu.VMEM((ERT, 1, 128), jnp.int32),
            pltpu.VMEM((ERT, 1, 128), jnp.int32),
            pltpu.VMEM((128, 128), jnp.float32),
            pltpu.VMEM((128, 128), jnp.float32),
            pltpu.VMEM((128, 128), jnp.float32),
            pltpu.VMEM_SHARED((NPH, 128), jnp.float32),
            pltpu.SemaphoreType.DMA,
            pltpu.SemaphoreType.DMA,
        ],
    )(_conv_sc_body)


def _conv_sc_body(hp0, hp1, hp2, hp3, si3d, di3d, o0, o1, o2, o3,
                  sidx_v, didx_v, dmap_v, rows0_v, rows1_v, zero_v, acc_sh,
                  sem0, sem1):
    hp = (hp0, hp1, hp2, hp3)
    outs = (o0, o1, o2, o3)

    c = lax.axis_index("c")
    s = lax.axis_index("s")

    def fz(i, _):
        for jj in range(8):
            zero_v[i, pl.ds(jj * 16, 16)] = jnp.zeros((16,), jnp.float32)
        return 0
    lax.fori_loop(0, 128, fz, 0)

    @pl.when(s < NS - 1)
    def _():
        pltpu.sync_copy(si3d.at[pl.ds(s * ERT, ERT)], sidx_v)
        pltpu.sync_copy(di3d.at[pl.ds(s * ERT, ERT)], didx_v)

    @pl.when(s == NS - 1)
    def _():
        pltpu.sync_copy(si3d.at[pl.ds((NS - 1) * ERT, ERLAST)],
                        sidx_v.at[pl.ds(0, ERLAST)])
        pltpu.sync_copy(di3d.at[pl.ds((NS - 1) * ERT, ERLAST)],
                        didx_v.at[pl.ds(0, ERLAST)])

    nr = jnp.where(s < NS - 1, ERT, ERLAST)

    def map_half(h):
        def mbody(i, _):
            row = i // 8
            lane = (i % 8) * 16
            v = didx_v[row, 0, pl.ds(lane, 16)]
            t = v - (h * NH)
            ok = (t >= 0) & (t < NH)
            dmap_v[row, 0, pl.ds(lane, 16)] = jnp.where(ok, t, DUMP)
            return 0
        lax.fori_loop(0, EL, mbody, 0)

    def run_pass(hp_g, out_g, h):
        for k in range(3):
            rcnt = min(128, NTH - k * 128)
            pltpu.sync_copy(zero_v.at[pl.ds(0, rcnt)],
                            acc_sh.at[pl.ds(s * NTH + k * 128, rcnt)])
        plsc.subcore_barrier()

        nr2 = nr // 2
        pltpu.async_copy(hp_g.at[sidx_v.at[0, 0]], rows0_v, sem0)

        def body(t, _):
            j0 = 2 * t
            pltpu.make_async_copy(hp_g.at[sidx_v.at[j0, 0]], rows0_v, sem0).wait()
            pltpu.async_copy(hp_g.at[sidx_v.at[j0 + 1, 0]], rows1_v, sem1)
            pltpu.sync_copy(rows0_v, acc_sh.at[dmap_v.at[j0, 0]], add=True)
            pltpu.make_async_copy(hp_g.at[sidx_v.at[j0 + 1, 0]], rows1_v, sem1).wait()

            @pl.when(t < nr2 - 1)
            def _():
                pltpu.async_copy(hp_g.at[sidx_v.at[j0 + 2, 0]], rows0_v, sem0)

            pltpu.sync_copy(rows1_v, acc_sh.at[dmap_v.at[j0 + 1, 0]], add=True)
            return 0
        lax.fori_loop(0, nr2, body, 0)

        plsc.subcore_barrier()
        for k in range(3):
            rcnt = min(128, NTH - k * 128)
            pltpu.sync_copy(acc_sh.at[pl.ds(s * NTH + k * 128, rcnt)],
                            rows0_v.at[pl.ds(0, rcnt)])
            pltpu.sync_copy(rows0_v.at[pl.ds(0, rcnt)],
                            out_g.at[pl.ds(h * NH + s * NTH + k * 128, rcnt)])
        plsc.subcore_barrier()

    for h in range(2):
        map_half(h)
        for p in range(2):
            @pl.when(c == 0)
            def _(p=p, h=h):
                run_pass(hp[p], outs[p], h)

            @pl.when(c == 1)
            def _(p=p, h=h):
                run_pass(hp[2 + p], outs[2 + p], h)


@functools.cache
def _deg2_sc_kernel():
    return functools.partial(
        pl.kernel, mesh=_mesh(),
        out_type=[jax.ShapeDtypeStruct((NP, 128), jnp.float32) for _ in range(2)],
        scratch_types=[
            pltpu.VMEM((DRW, 1, 128), jnp.int32),
            pltpu.VMEM((DRW, 1, 128), jnp.int32),
            pltpu.VMEM((128, 128), jnp.float32),
            pltpu.VMEM((128, 128), jnp.float32),
            pltpu.VMEM_SHARED((NPH, 128), jnp.float32),
        ],
    )(_deg2_sc_body)


def _deg2_sc_body(di3d, out0, out1, didx_v, dmap_v, ones_v, zero_v, acc_sh):
    outs = (out0, out1)
    c = lax.axis_index("c")
    s = lax.axis_index("s")
    w = c * NS + s

    def fill(i, _):
        for jj in range(8):
            zero_v[i, pl.ds(jj * 16, 16)] = jnp.zeros((16,), jnp.float32)
            ones_v[i, pl.ds(jj * 16, 16)] = jnp.ones((16,), jnp.float32)
        return 0
    lax.fori_loop(0, 128, fill, 0)

    @pl.when(w < 31)
    def _():
        pltpu.sync_copy(di3d.at[pl.ds(w * DRW, DRW)], didx_v)

    @pl.when(w == 31)
    def _():
        pltpu.sync_copy(di3d.at[pl.ds(31 * DRW, DRLAST)],
                        didx_v.at[pl.ds(0, DRLAST)])

    nr = jnp.where(w < 31, DRW, DRLAST)

    for h in range(2):
        def mbody(i, _, h=h):
            row = i // 8
            lane = (i % 8) * 16
            v = didx_v[row, 0, pl.ds(lane, 16)]
            t = v - (h * NH)
            ok = (t >= 0) & (t < NH)
            dmap_v[row, 0, pl.ds(lane, 16)] = jnp.where(ok, t, DUMP)
            return 0
        lax.fori_loop(0, DRW * 8, mbody, 0)

        for k in range(3):
            rcnt = min(128, NTH - k * 128)
            pltpu.sync_copy(zero_v.at[pl.ds(0, rcnt)],
                            acc_sh.at[pl.ds(s * NTH + k * 128, rcnt)])
        plsc.subcore_barrier()

        def body(j, _):
            pltpu.sync_copy(ones_v, acc_sh.at[dmap_v.at[j, 0]], add=True)
            return 0
        lax.fori_loop(0, nr, body, 0)

        plsc.subcore_barrier()
        for k in range(3):
            rcnt = min(128, NTH - k * 128)
            pltpu.sync_copy(acc_sh.at[pl.ds(s * NTH + k * 128, rcnt)],
                            zero_v.at[pl.ds(0, rcnt)])
            @pl.when(c == 0)
            def _(k=k, rcnt=rcnt, h=h):
                pltpu.sync_copy(zero_v.at[pl.ds(0, rcnt)],
                                outs[0].at[pl.ds(h * NH + s * NTH + k * 128, rcnt)])

            @pl.when(c == 1)
            def _(k=k, rcnt=rcnt, h=h):
                pltpu.sync_copy(zero_v.at[pl.ds(0, rcnt)],
                                outs[1].at[pl.ds(h * NH + s * NTH + k * 128, rcnt)])

        def refill(i, _):
            for jj in range(8):
                zero_v[i, pl.ds(jj * 16, 16)] = jnp.zeros((16,), jnp.float32)
            return 0
        lax.fori_loop(0, 128, refill, 0)
        plsc.subcore_barrier()


NPAD = 12288
ZR = NPAD // 128
ZRW = ZR // (NC * NS)
ZB = ZRW * 128


@functools.cache
def _zq_sc_kernel():
    return functools.partial(
        pl.kernel, mesh=_mesh(),
        out_type=jax.ShapeDtypeStruct((NPAD, CD), jnp.float32),
        scratch_types=[
            pltpu.VMEM((ZRW, 1, 128), jnp.int32),
            pltpu.VMEM((ZB, CD), jnp.float32),
            pltpu.SemaphoreType.DMA,
        ],
    )(_zq_sc_body)


def _zq_sc_body(cb_hbm, idx_hbm, out_hbm, idx_v, rows_v, sem):
    c = lax.axis_index("c")
    s = lax.axis_index("s")
    w = c * NS + s
    pltpu.sync_copy(idx_hbm.at[pl.ds(w * ZRW, ZRW)], idx_v)
    for j in range(ZRW):
        pltpu.async_copy(cb_hbm.at[idx_v.at[j, 0]],
                         rows_v.at[pl.ds(j * 128, 128)], sem).wait()
    pltpu.sync_copy(rows_v, out_hbm.at[pl.ds(w * ZB, ZB)])


MB = 400
KB = 512
NM = N // MB
NK = K // KB


def _vq_body(z_ref, cb_ref, out_ref, dmin_ref, amin_ref, cc_ref):
    k = pl.program_id(1)
    z = z_ref[...]
    cb = cb_ref[...]

    @pl.when(pl.program_id(0) == 0)
    def _():
        cc_ref[k, :] = jnp.sum(cb * cb, axis=1)

    zz = jnp.sum(z * z, axis=1, keepdims=True)
    prod = lax.dot_general(z, cb, (((1,), (1,)), ((), ())),
                           preferred_element_type=jnp.float32)
    d = zz - 2.0 * prod + cc_ref[k, :][None, :]
    loc_min = jnp.min(d, axis=1)
    loc_arg = jnp.argmin(d, axis=1).astype(jnp.int32) + k * KB

    @pl.when(k == 0)
    def _():
        dmin_ref[...] = loc_min
        amin_ref[...] = loc_arg

    @pl.when(k > 0)
    def _():
        upd = loc_min < dmin_ref[...]
        amin_ref[...] = jnp.where(upd, loc_arg, amin_ref[...])
        dmin_ref[...] = jnp.minimum(loc_min, dmin_ref[...])

    @pl.when(k == NK - 1)
    def _():
        out_ref[0, 0, :] = amin_ref[...]


def _vq_argmin(z, codebook):
    out = pl.pallas_call(
        _vq_body,
        grid=(NM, NK),
        in_specs=[
            pl.BlockSpec((MB, CD), lambda m, k: (m, 0)),
            pl.BlockSpec((KB, CD), lambda m, k: (k, 0)),
        ],
        out_specs=pl.BlockSpec((1, 1, MB), lambda m, k: (m, 0, 0)),
        out_shape=jax.ShapeDtypeStruct((NM, 1, MB), jnp.int32),
        scratch_shapes=[
            pltpu.VMEM((MB,), jnp.float32),
            pltpu.VMEM((MB,), jnp.int32),
            pltpu.VMEM((NK, KB), jnp.float32),
        ],
    )(z, codebook)
    return out.reshape(N)


MBLK = 1000
GRID = N // MBLK


def _full(shape):
    return pl.BlockSpec(shape, lambda i: tuple(0 for _ in shape))


def _rows(width):
    return pl.BlockSpec((MBLK, width), lambda i: (i, 0))


def _gout(width=128):
    return pl.BlockSpec((MBLK, width), lambda i: (i, 0))


def _gcn_combine(acc_refs, hp_refs, dinv, b):
    a = jnp.concatenate(
        [acc_refs[g][...] + hp_refs[g][...] for g in range(len(acc_refs))], axis=1)
    return jnp.maximum(a * dinv + b, 0.0)


def _enc1_body(x_ref, dinv_ref, w_ref, o0, o1, o2, o3):
    y = jnp.dot(x_ref[...], w_ref[...], preferred_element_type=jnp.float32)
    y = y * dinv_ref[...]
    for g, o in enumerate((o0, o1, o2, o3)):
        o[...] = y[:, g * 128:(g + 1) * 128]


def _enc1(x, dinv, W1):
    return pl.pallas_call(
        _enc1_body,
        grid=(GRID,),
        in_specs=[_rows(IN), _rows(1), _full((IN, H))],
        out_specs=[_gout() for _ in range(4)],
        out_shape=[jax.ShapeDtypeStruct((N, 128), jnp.float32) for _ in range(4)],
    )(x, dinv, W1)


def _mid_body(a0, a1, a2, a3, h0, h1, h2, h3, dinv_ref, b_ref, w_ref,
              o0, o1, o2, o3):
    z = _gcn_combine((a0, a1, a2, a3), (h0, h1, h2, h3), dinv_ref[...], b_ref[...])
    y = jnp.dot(z, w_ref[...], preferred_element_type=jnp.float32)
    y = y * dinv_ref[...]
    for g, o in enumerate((o0, o1, o2, o3)):
        o[...] = y[:, g * 128:(g + 1) * 128]


def _mid(accs, hps, dinv, b, W):
    return pl.pallas_call(
        _mid_body,
        grid=(GRID,),
        in_specs=[_gout() for _ in range(8)] + [_rows(1), _full((1, H)), _full((H, H))],
        out_specs=[_gout() for _ in range(4)],
        out_shape=[jax.ShapeDtypeStruct((N, 128), jnp.float32) for _ in range(4)],
    )(*accs, *hps, dinv, b, W)


def _enc3_body(a0, a1, a2, a3, h0, h1, h2, h3, dinv_ref, b_ref, wc_ref, bc_ref, o):
    z2 = _gcn_combine((a0, a1, a2, a3), (h0, h1, h2, h3), dinv_ref[...], b_ref[...])
    o[...] = jnp.dot(z2, wc_ref[...], preferred_element_type=jnp.float32) + bc_ref[...]


def _enc3(accs, hps, dinv, b, Wc, bc):
    return pl.pallas_call(
        _enc3_body,
        grid=(GRID,),
        in_specs=[_gout() for _ in range(8)]
        + [_rows(1), _full((1, H)), _full((H, CD)), _full((1, CD))],
        out_specs=_rows(CD),
        out_shape=jax.ShapeDtypeStruct((N, CD), jnp.float32),
    )(*accs, *hps, dinv, b, Wc, bc)


def _dec1_body(zq_ref, z_ref, wf_ref, bf_ref, wd1_ref, dinv_ref,
               o0, o1, o2, o3, ls_ref):
    zq = zq_ref[...]
    h = jnp.dot(zq, wf_ref[...], preferred_element_type=jnp.float32) + bf_ref[...]
    y = jnp.dot(h, wd1_ref[...], preferred_element_type=jnp.float32)
    y = y * dinv_ref[...]
    for g, o in enumerate((o0, o1, o2, o3)):
        o[...] = y[:, g * 128:(g + 1) * 128]
    diff = zq - z_ref[...]
    part = jnp.sum(diff * diff)

    @pl.when(pl.program_id(0) == 0)
    def _():
        ls_ref[...] = part.reshape(1, 1)

    @pl.when(pl.program_id(0) > 0)
    def _():
        ls_ref[...] = ls_ref[...] + part.reshape(1, 1)


def _dec1(zq, z, Wf, bf, Wd1, dinv):
    return pl.pallas_call(
        _dec1_body,
        grid=(GRID,),
        in_specs=[_rows(CD), _rows(CD), _full((CD, H)), _full((1, H)),
                  _full((H, H)), _rows(1)],
        out_specs=[_gout() for _ in range(4)]
        + [pl.BlockSpec((1, 1), lambda i: (0, 0))],
        out_shape=[jax.ShapeDtypeStruct((N, 128), jnp.float32) for _ in range(4)]
        + [jax.ShapeDtypeStruct((1, 1), jnp.float32)],
    )(zq, z, Wf, bf, Wd1, dinv)


def _dec2_body(a0, a1, a2, a3, h0, h1, h2, h3, dinv_ref, b_ref, w_ref, o0, o1):
    h3v = _gcn_combine((a0, a1, a2, a3), (h0, h1, h2, h3), dinv_ref[...], b_ref[...])
    y = jnp.dot(h3v, w_ref[...], preferred_element_type=jnp.float32)
    y = y * dinv_ref[...]
    for g, o in enumerate((o0, o1)):
        o[...] = y[:, g * 128:(g + 1) * 128]


def _dec2(accs, hps, dinv, b, Wd2):
    return pl.pallas_call(
        _dec2_body,
        grid=(GRID,),
        in_specs=[_gout() for _ in range(8)] + [_rows(1), _full((1, H)), _full((H, IN))],
        out_specs=[_gout() for _ in range(2)],
        out_shape=[jax.ShapeDtypeStruct((N, 128), jnp.float32) for _ in range(2)],
    )(*accs, *hps, dinv, b, Wd2)


def _dec3_body(a0, a1, h0, h1, dinv_ref, b_ref, o):
    a = jnp.concatenate([a0[...] + h0[...], a1[...] + h1[...]], axis=1)
    o[...] = a * dinv_ref[...] + b_ref[...]


def _dec3(accs, hps, dinv, b):
    return pl.pallas_call(
        _dec3_body,
        grid=(GRID,),
        in_specs=[_gout() for _ in range(4)] + [_rows(1), _full((1, IN))],
        out_specs=_rows(IN),
        out_shape=jax.ShapeDtypeStruct((N, IN), jnp.float32),
    )(*accs, *hps, dinv, b)


def kernel(x, edge_index, W1, b1, W2, b2, Wc, bc, codebook, Wf, bf, Wd1, bd1, Wd2, bd2):
    si3d = edge_index[0].reshape(ER, 1, 128)
    di3d = edge_index[1].reshape(ER, 1, 128)

    conv = _conv_sc_kernel()

    d0, d1 = _deg2_sc_kernel()(di3d)
    deg = d0[:N, 0] + d1[:N, 0] + 1.0
    dinv = lax.rsqrt(deg).reshape(N, 1)

    hp1 = _enc1(x, dinv, W1)
    acc1 = conv(*hp1, si3d, di3d)
    hp2 = _mid(acc1, hp1, dinv, b1.reshape(1, H), W2)
    acc2 = conv(*hp2, si3d, di3d)
    z = _enc3(acc2, hp2, dinv, b2.reshape(1, H), Wc, bc.reshape(1, CD))

    indices = _vq_argmin(z, codebook)

    idxp = jnp.pad(indices, (0, NPAD - N)).reshape(ZR, 1, 128)
    z_q = _zq_sc_kernel()(codebook, idxp)[:N]

    *hp3, loss_sum = _dec1(z_q, z, Wf, bf.reshape(1, H), Wd1, dinv)
    loss = loss_sum[0, 0] * ((1.0 + CC) / (N * CD))

    acc3 = conv(*hp3, si3d, di3d)
    hp4 = _dec2(acc3, hp3, dinv, bd1.reshape(1, H), Wd2)
    acc4 = conv(hp4[0], hp4[1], hp4[0], hp4[1], si3d, di3d)[:2]
    x_recon = _dec3(acc4, hp4, dinv, bd2.reshape(1, IN))

    return (x_recon, loss, indices)
